# Initial kernel scaffold; baseline (speedup 1.0000x reference)
import jax
import jax.numpy as jnp
from jax.experimental import pallas as pl


def kernel(x, edge_index, W1_l, W1_r, b1, W2_l, W2_r, b2):
    raise NotImplementedError("write your pallas kernel here")


# speedup vs baseline: 13.5997x; 13.5997x over previous
import functools

import jax
import jax.numpy as jnp
from jax import lax
from jax.experimental import pallas as pl
from jax.experimental.pallas import tpu as pltpu
from jax.experimental.pallas import tpu_sc as plsc

N_NODES = 10000
N_PAD = 10240
D0 = 128
DH = 256
NC = 2
NS = 16
NW = NC * NS
CHUNK = 100
ZROWS = 80


def _make_seg_sum(nchunk, with_count):
    mesh = plsc.VectorSubcoreMesh(core_axis_name="c", subcore_axis_name="s")
    rows_per_tile = N_PAD // NS

    out_type = [jax.ShapeDtypeStruct((NC, N_PAD, D0), jnp.float32)]
    scratch = [
        pltpu.VMEM((nchunk, CHUNK), jnp.int32),
        pltpu.VMEM((CHUNK,), jnp.int32),
        pltpu.VMEM((CHUNK,), jnp.int32),
        pltpu.VMEM((CHUNK, D0), jnp.float32),
        pltpu.VMEM((CHUNK, D0), jnp.float32),
        pltpu.VMEM_SHARED((N_PAD, D0), jnp.float32),
        pltpu.SemaphoreType.DMA,
        pltpu.SemaphoreType.DMA,
        pltpu.SemaphoreType.DMA,
        pltpu.SemaphoreType.DMA,
    ]
    if with_count:
        out_type.append(jax.ShapeDtypeStruct((NC, N_PAD), jnp.float32))
        scratch += [
            pltpu.VMEM((128,), jnp.float32),
            pltpu.VMEM((128,), jnp.float32),
            pltpu.VMEM_SHARED((N_PAD,), jnp.float32),
        ]

    def body(*refs):
        if with_count:
            (table, sidx_h, didx_h, sum_out, cnt_out,
             didx_v, sidx_a, sidx_b, rows_a, rows_b, acc,
             isem_a, isem_b, rsem_a, rsem_b, ones_v, zer_v, cntacc) = refs
        else:
            (table, sidx_h, didx_h, sum_out,
             didx_v, sidx_a, sidx_b, rows_a, rows_b, acc,
             isem_a, isem_b, rsem_a, rsem_b) = refs

        cid = lax.axis_index("c")
        sid = lax.axis_index("s")
        wid = cid * NS + sid

        pltpu.sync_copy(didx_h.at[wid], didx_v)

        zero16 = jnp.zeros((16,), jnp.float32)

        def _zrow(i, carry):
            for c in range(D0 // 16):
                rows_a[i, pl.ds(c * 16, 16)] = zero16
            return carry

        lax.fori_loop(0, ZROWS, _zrow, 0)
        base = sid * rows_per_tile
        for k in range(rows_per_tile // ZROWS):
            pltpu.sync_copy(rows_a.at[pl.ds(0, ZROWS)],
                            acc.at[pl.ds(base + k * ZROWS, ZROWS)])
        if with_count:
            for c in range(128 // 16):
                ones_v[pl.ds(c * 16, 16)] = jnp.ones((16,), jnp.float32)
                zer_v[pl.ds(c * 16, 16)] = zero16
            for k in range(rows_per_tile // ZROWS):
                pltpu.sync_copy(zer_v.at[pl.ds(0, ZROWS)],
                                cntacc.at[pl.ds(base + k * ZROWS, ZROWS)])

        plsc.subcore_barrier()

        def _idx_cp(j, sbuf, sem):
            return pltpu.make_async_copy(sidx_h.at[wid, j], sbuf, sem)

        def _row_cp(sbuf, rbuf, sem):
            return pltpu.make_async_copy(table.at[sbuf], rbuf, sem)

        _idx_cp(0, sidx_a, isem_a).start()
        _idx_cp(1, sidx_b, isem_b).start()
        _idx_cp(0, sidx_a, isem_a).wait()
        _row_cp(sidx_a, rows_a, rsem_a).start()

        def _do(j, s_m, i_m, r_m, rs_m, s_o, i_o, r_o, rs_o):
            @pl.when(j + 1 < nchunk)
            def _():
                _idx_cp(j + 1, s_o, i_o).wait()
                _row_cp(s_o, r_o, rs_o).start()

            _row_cp(s_m, r_m, rs_m).wait()
            pltpu.sync_copy(r_m, acc.at[didx_v.at[j]], add=True)
            if with_count:
                pltpu.sync_copy(ones_v.at[pl.ds(0, CHUNK)],
                                cntacc.at[didx_v.at[j]], add=True)

            @pl.when(j + 2 < nchunk)
            def _():
                _idx_cp(j + 2, s_m, i_m).start()

        def _pair(i, carry):
            j = 2 * i
            _do(j, sidx_a, isem_a, rows_a, rsem_a,
                sidx_b, isem_b, rows_b, rsem_b)
            _do(j + 1, sidx_b, isem_b, rows_b, rsem_b,
                sidx_a, isem_a, rows_a, rsem_a)
            return carry

        lax.fori_loop(0, nchunk // 2, _pair, 0)
        if nchunk % 2:
            _do(nchunk - 1, sidx_a, isem_a, rows_a, rsem_a,
                sidx_b, isem_b, rows_b, rsem_b)

        plsc.subcore_barrier()

        pltpu.sync_copy(acc.at[pl.ds(base, rows_per_tile)],
                        sum_out.at[cid, pl.ds(base, rows_per_tile)])
        if with_count:
            pltpu.sync_copy(cntacc.at[pl.ds(base, rows_per_tile)],
                            cnt_out.at[cid, pl.ds(base, rows_per_tile)])

    return pl.kernel(body, out_type=out_type, mesh=mesh, scratch_types=scratch)


_DN = (((1,), (1,)), ((), ()))


def _tc1_body(ps1, cnt, x, w1l, w1r, b1, w2l, w2r, b2, y2_o, hr_o):
    s = ps1[0] + ps1[1]
    c = cnt[0] + cnt[1]
    r = 1.0 / jnp.maximum(c, 1.0)
    mean = s * r[:, None]
    h = lax.dot_general(mean, w1l[...], _DN, preferred_element_type=jnp.float32)
    h += lax.dot_general(x[...], w1r[...], _DN, preferred_element_type=jnp.float32)
    h = jnp.maximum(h + b1[...][None, :], 0.0)
    y2_o[...] = lax.dot_general(h, w2l[...], _DN, preferred_element_type=jnp.float32)
    hr_o[...] = (lax.dot_general(h, w2r[...], _DN, preferred_element_type=jnp.float32)
                 + b2[...][None, :])


def _tc2_body(ps2, cnt, hr, out):
    s = ps2[0] + ps2[1]
    c = cnt[0] + cnt[1]
    r = 1.0 / jnp.maximum(c, 1.0)
    out[...] = s * r[:, None] + hr[...]


def kernel(x, edge_index, W1_l, W1_r, b1, W2_l, W2_r, b2):
    n = x.shape[0]
    e = edge_index.shape[1]
    epw = e // NW
    nchunk = epw // CHUNK
    src = edge_index[0].astype(jnp.int32).reshape(NW, nchunk, CHUNK)
    dst = edge_index[1].astype(jnp.int32).reshape(NW, nchunk, CHUNK)
    x_pad = jnp.zeros((N_PAD, D0), jnp.float32).at[:n].set(x)

    seg1 = _make_seg_sum(nchunk, with_count=True)
    seg2 = _make_seg_sum(nchunk, with_count=False)

    ps1, pcnt = seg1(x_pad, src, dst)

    bm = 1280
    grid = (N_PAD // bm,)
    y2, hr = pl.pallas_call(
        _tc1_body,
        grid=grid,
        in_specs=[
            pl.BlockSpec((NC, bm, D0), lambda i: (0, i, 0)),
            pl.BlockSpec((NC, bm), lambda i: (0, i)),
            pl.BlockSpec((bm, D0), lambda i: (i, 0)),
            pl.BlockSpec((DH, D0), lambda i: (0, 0)),
            pl.BlockSpec((DH, D0), lambda i: (0, 0)),
            pl.BlockSpec((DH,), lambda i: (0,)),
            pl.BlockSpec((D0, DH), lambda i: (0, 0)),
            pl.BlockSpec((D0, DH), lambda i: (0, 0)),
            pl.BlockSpec((D0,), lambda i: (0,)),
        ],
        out_specs=[
            pl.BlockSpec((bm, D0), lambda i: (i, 0)),
            pl.BlockSpec((bm, D0), lambda i: (i, 0)),
        ],
        out_shape=[
            jax.ShapeDtypeStruct((N_PAD, D0), jnp.float32),
            jax.ShapeDtypeStruct((N_PAD, D0), jnp.float32),
        ],
    )(ps1, pcnt, x_pad, W1_l, W1_r, b1, W2_l, W2_r, b2)

    (ps2,) = seg2(y2, src, dst)

    out = pl.pallas_call(
        _tc2_body,
        grid=grid,
        in_specs=[
            pl.BlockSpec((NC, bm, D0), lambda i: (0, i, 0)),
            pl.BlockSpec((NC, bm), lambda i: (0, i)),
            pl.BlockSpec((bm, D0), lambda i: (i, 0)),
        ],
        out_specs=pl.BlockSpec((bm, D0), lambda i: (i, 0)),
        out_shape=jax.ShapeDtypeStruct((N_PAD, D0), jnp.float32),
    )(ps2, pcnt, hr)

    return out[:n]
